# baseline (speedup 1.0000x reference)
import functools

import jax
import jax.numpy as jnp
from jax import lax
from jax.experimental import pallas as pl
from jax.experimental.pallas import tpu as pltpu
from jax.experimental.pallas import tpu_sc as plsc

N = 10000
NP = 10240
E = 160000
D = 256
H = 128
NC = 2
NS = 16
EP = 163840
EPS = EP // NS
C = 128
NCHUNK = EPS // C
RPS = NP // NS
ZR = 64
DW = 16

_mesh = plsc.VectorSubcoreMesh(core_axis_name="c", subcore_axis_name="s",
                               num_cores=NC, num_subcores=NS)


def _sc_agg_body(xs, srcs, dstp, summed,
                 srcidx0, srcidx1, dstidx0, dstidx1, rows0, rows1,
                 stage_v, semg, sems0, sems1, acc):
    c = lax.axis_index("c")
    s = lax.axis_index("s")

    srcidx = (srcidx0, srcidx1)
    dstidx = (dstidx0, dstidx1)
    rows = (rows0, rows1)
    sems = (sems0, sems1)

    zero16 = jnp.zeros((16,), jnp.float32)

    def fill(i, _):
        for j in range(H // 16):
            stage_v[i, pl.ds(j * 16, 16)] = zero16
        return 0
    lax.fori_loop(0, ZR, fill, 0)

    row0 = s * RPS
    for t in range(RPS // ZR):
        pltpu.sync_copy(stage_v, acc.at[pl.ds(row0 + t * ZR, ZR)])
    plsc.subcore_barrier()

    ebase = s * EPS

    def half(k, b):
        @pl.when(k >= 2)
        def _():
            pltpu.make_async_copy(xs.at[pl.ds(0, C)], rows[b], sems[b]).wait()
        off = ebase + k * C
        pltpu.sync_copy(srcs.at[pl.ds(c * EP + off, C)], srcidx[b])
        pltpu.sync_copy(dstp.at[pl.ds(off, C)], dstidx[b])
        pltpu.async_copy(xs.at[srcidx[b]], rows[b], semg).wait()
        pltpu.async_copy(rows[b], acc.at[dstidx[b]], sems[b], add=True)

    def chunk2(k2, _):
        half(k2 * 2, 0)
        half(k2 * 2 + 1, 1)
        return 0
    lax.fori_loop(0, NCHUNK // 2, chunk2, 0)
    for b in range(2):
        pltpu.make_async_copy(xs.at[pl.ds(0, C)], rows[b], sems[b]).wait()
    plsc.subcore_barrier()

    for t in range(RPS // ZR):
        r = row0 + t * ZR
        pltpu.sync_copy(acc.at[pl.ds(r, ZR)], stage_v)
        pltpu.sync_copy(stage_v, summed.at[pl.ds(c * NP + r, ZR)])


_sc_agg = pl.kernel(
    _sc_agg_body,
    out_type=(jax.ShapeDtypeStruct((NC * NP, H), jnp.float32),),
    mesh=_mesh,
    scratch_types=[
        pltpu.VMEM((C,), jnp.int32),
        pltpu.VMEM((C,), jnp.int32),
        pltpu.VMEM((C,), jnp.int32),
        pltpu.VMEM((C,), jnp.int32),
        pltpu.VMEM((C, H), jnp.float32),
        pltpu.VMEM((C, H), jnp.float32),
        pltpu.VMEM((ZR, H), jnp.float32),
        pltpu.SemaphoreType.DMA,
        pltpu.SemaphoreType.DMA,
        pltpu.SemaphoreType.DMA,
        pltpu.VMEM_SHARED((NP, H), jnp.float32),
    ],
)


QR = NP // 8
EPD = EP // (NC * NS)
NCHD = EPD // C
DRPS = QR // NS


def _sc_deg_body(dstp, onest, deg, dstidx_v, qidx_v, rows_v, sem, degacc):
    c = lax.axis_index("c")
    s = lax.axis_index("s")

    zero16 = jnp.zeros((16,), jnp.float32)

    def fill(i, _):
        for j in range(H // 16):
            rows_v[i, pl.ds(j * 16, 16)] = zero16
        return 0
    lax.fori_loop(0, C, fill, 0)

    row0 = s * DRPS
    pltpu.sync_copy(rows_v.at[pl.ds(0, DRPS)], degacc.at[pl.ds(row0, DRPS)])
    plsc.subcore_barrier()

    ebase = (c * NS + s) * EPD
    def chunk(k, _):
        pltpu.sync_copy(dstp.at[pl.ds(ebase + k * C, C)], dstidx_v)
        for j in range(C // 16):
            d = dstidx_v[pl.ds(j * 16, 16)]
            qidx_v[pl.ds(j * 16, 16)] = jax.lax.shift_right_logical(d, 3)
            dstidx_v[pl.ds(j * 16, 16)] = jnp.bitwise_and(d, 7)
        pltpu.async_copy(onest.at[dstidx_v], rows_v, sem).wait()
        pltpu.sync_copy(rows_v, degacc.at[qidx_v], add=True)
        return 0
    lax.fori_loop(0, NCHD, chunk, 0)
    plsc.subcore_barrier()

    pltpu.sync_copy(degacc.at[pl.ds(row0, DRPS)], rows_v.at[pl.ds(0, DRPS)])
    pltpu.sync_copy(rows_v.at[pl.ds(0, DRPS)], deg.at[pl.ds(c * QR + row0, DRPS)])


_sc_deg = pl.kernel(
    _sc_deg_body,
    out_type=(jax.ShapeDtypeStruct((NC * QR, H), jnp.float32),),
    mesh=_mesh,
    scratch_types=[
        pltpu.VMEM((C,), jnp.int32),
        pltpu.VMEM((C,), jnp.int32),
        pltpu.VMEM((C, H), jnp.float32),
        pltpu.SemaphoreType.DMA,
        pltpu.VMEM_SHARED((QR, H), jnp.float32),
    ],
)


R = 512
G = NP // R


def _tc_body(xa, xb, sa, sb, d0, d1, ws, wn, b, flag, o1, o2):
    dsum = d0[...] + d1[...]
    degn = dsum.reshape(R // 8, 8, 16)[:, :, 0].reshape(R, 1)
    invd = 1.0 / jnp.maximum(degn, 1.0)
    f32 = jnp.float32
    h = (jnp.dot(xa[...], ws[0:H, :], preferred_element_type=f32)
         + jnp.dot(xb[...], ws[H:D, :], preferred_element_type=f32)
         + jnp.dot(sa[...] * invd, wn[0:H, :], preferred_element_type=f32)
         + jnp.dot(sb[...] * invd, wn[H:D, :], preferred_element_type=f32)
         + b[...])
    h = jnp.where(flag[0, 0] > 0.5, jnp.maximum(h, 0.0), h)
    o1[...] = h[:, 0:H]
    o2[...] = h[:, H:D]


_tc_layer = pl.pallas_call(
    _tc_body,
    grid=(G,),
    in_specs=[
        pl.BlockSpec((R, H), lambda i: (i, 0)),
        pl.BlockSpec((R, H), lambda i: (i + G, 0)),
        pl.BlockSpec((R, H), lambda i: (i, 0)),
        pl.BlockSpec((R, H), lambda i: (i + G, 0)),
        pl.BlockSpec((R // 8, H), lambda i: (i, 0)),
        pl.BlockSpec((R // 8, H), lambda i: (i + G, 0)),
        pl.BlockSpec((D, D), lambda i: (0, 0)),
        pl.BlockSpec((D, D), lambda i: (0, 0)),
        pl.BlockSpec((1, D), lambda i: (0, 0)),
        pl.BlockSpec((1, 1), lambda i: (0, 0)),
    ],
    out_specs=(pl.BlockSpec((R, H), lambda i: (i, 0)),
               pl.BlockSpec((R, H), lambda i: (i, 0))),
    out_shape=(jax.ShapeDtypeStruct((NP, H), jnp.float32),
               jax.ShapeDtypeStruct((NP, H), jnp.float32)),
)


@jax.jit
def kernel(x, edge_index, W_self1, W_neigh1, b1, W_self2, W_neigh2, b2):
    xp = jnp.pad(x, ((0, NP - N), (0, 0)))
    xs = xp.reshape(NP, NC, H).transpose(1, 0, 2).reshape(NC * NP, H)
    src = edge_index[0]
    dst = edge_index[1]
    srcp = jnp.pad(src, (0, EP - E), constant_values=NP - 1)
    srcs = jnp.concatenate([srcp, srcp + NP]).astype(jnp.int32)
    dstp = jnp.pad(dst, (0, EP - E), constant_values=NP - 1).astype(jnp.int32)
    onest = jnp.repeat(jnp.eye(8, dtype=jnp.float32), H // 8, axis=1)
    (deg,) = _sc_deg(dstp, onest)

    wss = jnp.stack([W_self1, W_self2])
    wns = jnp.stack([W_neigh1, W_neigh2])
    bss = jnp.stack([b1.reshape(1, D), b2.reshape(1, D)])
    flags = jnp.array([[[1.0]], [[0.0]]], dtype=jnp.float32)

    def body(hs, per):
        wsi, wni, bi, fl = per
        (summed,) = _sc_agg(hs, srcs, dstp)
        o1, o2 = _tc_layer(hs, hs, summed, summed, deg, deg, wsi, wni, bi, fl)
        return jnp.concatenate([o1, o2], axis=0), None

    hs_final, _ = lax.scan(body, xs, (wss, wns, bss, flags))
    return jnp.concatenate([hs_final[:N], hs_final[NP:NP + N]], axis=1)


# speedup vs baseline: 1.9399x; 1.9399x over previous
import functools

import jax
import jax.numpy as jnp
from jax import lax
from jax.experimental import pallas as pl
from jax.experimental.pallas import tpu as pltpu
from jax.experimental.pallas import tpu_sc as plsc

N = 10000
NP = 10240
E = 160000
D = 256
H = 128
NC = 2
NS = 16
EP = 163840
EPS = EP // NS
C = 128
NCHUNK = EPS // C
RPS = NP // NS
ZR = 64
DW = 16

_mesh = plsc.VectorSubcoreMesh(core_axis_name="c", subcore_axis_name="s",
                               num_cores=NC, num_subcores=NS)


def _sc_agg_body(xs, srcs, dstp, summed,
                 srcidx0, srcidx1, dstidx0, dstidx1, rows0, rows1,
                 stage_v, semg, sems0, sems1, acc):
    c = lax.axis_index("c")
    s = lax.axis_index("s")

    srcidx = (srcidx0, srcidx1)
    dstidx = (dstidx0, dstidx1)
    rows = (rows0, rows1)
    sems = (sems0, sems1)

    zero16 = jnp.zeros((16,), jnp.float32)

    def fill(i, _):
        for j in range(H // 16):
            stage_v[i, pl.ds(j * 16, 16)] = zero16
        return 0
    lax.fori_loop(0, ZR, fill, 0)

    row0 = s * RPS
    for t in range(RPS // ZR):
        pltpu.sync_copy(stage_v, acc.at[pl.ds(row0 + t * ZR, ZR)])
    plsc.subcore_barrier()

    ebase = s * EPS

    def half(k, b):
        @pl.when(k >= 2)
        def _():
            pltpu.make_async_copy(xs.at[pl.ds(0, C)], rows[b], sems[b]).wait()
        off = ebase + k * C
        pltpu.sync_copy(srcs.at[pl.ds(c * EP + off, C)], srcidx[b])
        pltpu.sync_copy(dstp.at[pl.ds(off, C)], dstidx[b])
        pltpu.async_copy(xs.at[srcidx[b]], rows[b], semg).wait()
        pltpu.async_copy(rows[b], acc.at[dstidx[b]], sems[b], add=True)

    def chunk2(k2, _):
        half(k2 * 2, 0)
        half(k2 * 2 + 1, 1)
        return 0
    lax.fori_loop(0, NCHUNK // 2, chunk2, 0)
    for b in range(2):
        pltpu.make_async_copy(xs.at[pl.ds(0, C)], rows[b], sems[b]).wait()
    plsc.subcore_barrier()

    for t in range(RPS // ZR):
        r = row0 + t * ZR
        pltpu.sync_copy(acc.at[pl.ds(r, ZR)], stage_v)
        pltpu.sync_copy(stage_v, summed.at[pl.ds(c * NP + r, ZR)])


_sc_agg = pl.kernel(
    _sc_agg_body,
    out_type=(jax.ShapeDtypeStruct((NC * NP, H), jnp.float32),),
    mesh=_mesh,
    scratch_types=[
        pltpu.VMEM((C,), jnp.int32),
        pltpu.VMEM((C,), jnp.int32),
        pltpu.VMEM((C,), jnp.int32),
        pltpu.VMEM((C,), jnp.int32),
        pltpu.VMEM((C, H), jnp.float32),
        pltpu.VMEM((C, H), jnp.float32),
        pltpu.VMEM((ZR, H), jnp.float32),
        pltpu.SemaphoreType.DMA,
        pltpu.SemaphoreType.DMA,
        pltpu.SemaphoreType.DMA,
        pltpu.VMEM_SHARED((NP, H), jnp.float32),
    ],
)


QR = NP // 8
EPD = EP // (NC * NS)
NCHD = EPD // C
DRPS = QR // NS


def _sc_deg_body(dstp, onest, deg,
                 dstidx0, dstidx1, qidx0, qidx1, rows0, rows1,
                 semg, sems0, sems1, degacc, onessp):
    c = lax.axis_index("c")
    s = lax.axis_index("s")

    dstidx = (dstidx0, dstidx1)
    qidx = (qidx0, qidx1)
    rows = (rows0, rows1)
    sems = (sems0, sems1)

    zero16 = jnp.zeros((16,), jnp.float32)

    def fill(i, _):
        for j in range(H // 16):
            rows0[i, pl.ds(j * 16, 16)] = zero16
        return 0
    lax.fori_loop(0, C, fill, 0)

    row0 = s * DRPS
    pltpu.sync_copy(rows0.at[pl.ds(0, DRPS)], degacc.at[pl.ds(row0, DRPS)])
    @pl.when(s == 0)
    def _():
        pltpu.sync_copy(onest, onessp)
    plsc.subcore_barrier()

    ebase = (c * NS + s) * EPD

    def half(k, b):
        @pl.when(k >= 2)
        def _():
            pltpu.make_async_copy(deg.at[pl.ds(0, C)], rows[b], sems[b]).wait()
        pltpu.sync_copy(dstp.at[pl.ds(ebase + k * C, C)], dstidx[b])
        for j in range(C // 16):
            d = dstidx[b][pl.ds(j * 16, 16)]
            qidx[b][pl.ds(j * 16, 16)] = jax.lax.shift_right_logical(d, 3)
            dstidx[b][pl.ds(j * 16, 16)] = jnp.bitwise_and(d, 7)
        pltpu.async_copy(onessp.at[dstidx[b]], rows[b], semg).wait()
        pltpu.async_copy(rows[b], degacc.at[qidx[b]], sems[b], add=True)

    def chunk2(k2, _):
        half(k2 * 2, 0)
        half(k2 * 2 + 1, 1)
        return 0
    lax.fori_loop(0, NCHD // 2, chunk2, 0)
    for b in range(2):
        pltpu.make_async_copy(deg.at[pl.ds(0, C)], rows[b], sems[b]).wait()
    plsc.subcore_barrier()

    pltpu.sync_copy(degacc.at[pl.ds(row0, DRPS)], rows0.at[pl.ds(0, DRPS)])
    pltpu.sync_copy(rows0.at[pl.ds(0, DRPS)], deg.at[pl.ds(c * QR + row0, DRPS)])


_sc_deg = pl.kernel(
    _sc_deg_body,
    out_type=(jax.ShapeDtypeStruct((NC * QR, H), jnp.float32),),
    mesh=_mesh,
    scratch_types=[
        pltpu.VMEM((C,), jnp.int32),
        pltpu.VMEM((C,), jnp.int32),
        pltpu.VMEM((C,), jnp.int32),
        pltpu.VMEM((C,), jnp.int32),
        pltpu.VMEM((C, H), jnp.float32),
        pltpu.VMEM((C, H), jnp.float32),
        pltpu.SemaphoreType.DMA,
        pltpu.SemaphoreType.DMA,
        pltpu.SemaphoreType.DMA,
        pltpu.VMEM_SHARED((QR, H), jnp.float32),
        pltpu.VMEM_SHARED((8, H), jnp.float32),
    ],
)


R = 512
G = NP // R


def _tc_body(xa, xb, sa, sb, d0, d1, ws, wn, b, flag, o1, o2):
    dsum = d0[...] + d1[...]
    degn = dsum.reshape(R // 8, 8, 16)[:, :, 0].reshape(R, 1)
    invd = 1.0 / jnp.maximum(degn, 1.0)
    f32 = jnp.float32
    h = (jnp.dot(xa[...], ws[0:H, :], preferred_element_type=f32)
         + jnp.dot(xb[...], ws[H:D, :], preferred_element_type=f32)
         + jnp.dot(sa[...] * invd, wn[0:H, :], preferred_element_type=f32)
         + jnp.dot(sb[...] * invd, wn[H:D, :], preferred_element_type=f32)
         + b[...])
    h = jnp.where(flag[0, 0] > 0.5, jnp.maximum(h, 0.0), h)
    o1[...] = h[:, 0:H]
    o2[...] = h[:, H:D]


_tc_layer = pl.pallas_call(
    _tc_body,
    grid=(G,),
    in_specs=[
        pl.BlockSpec((R, H), lambda i: (i, 0)),
        pl.BlockSpec((R, H), lambda i: (i + G, 0)),
        pl.BlockSpec((R, H), lambda i: (i, 0)),
        pl.BlockSpec((R, H), lambda i: (i + G, 0)),
        pl.BlockSpec((R // 8, H), lambda i: (i, 0)),
        pl.BlockSpec((R // 8, H), lambda i: (i + G, 0)),
        pl.BlockSpec((D, D), lambda i: (0, 0)),
        pl.BlockSpec((D, D), lambda i: (0, 0)),
        pl.BlockSpec((1, D), lambda i: (0, 0)),
        pl.BlockSpec((1, 1), lambda i: (0, 0)),
    ],
    out_specs=(pl.BlockSpec((R, H), lambda i: (i, 0)),
               pl.BlockSpec((R, H), lambda i: (i, 0))),
    out_shape=(jax.ShapeDtypeStruct((NP, H), jnp.float32),
               jax.ShapeDtypeStruct((NP, H), jnp.float32)),
)


@jax.jit
def kernel(x, edge_index, W_self1, W_neigh1, b1, W_self2, W_neigh2, b2):
    xp = jnp.pad(x, ((0, NP - N), (0, 0)))
    xs = xp.reshape(NP, NC, H).transpose(1, 0, 2).reshape(NC * NP, H)
    src = edge_index[0]
    dst = edge_index[1]
    srcp = jnp.pad(src, (0, EP - E), constant_values=NP - 1)
    srcs = jnp.concatenate([srcp, srcp + NP]).astype(jnp.int32)
    dstp = jnp.pad(dst, (0, EP - E), constant_values=NP - 1).astype(jnp.int32)
    onest = jnp.repeat(jnp.eye(8, dtype=jnp.float32), H // 8, axis=1)
    (deg,) = _sc_deg(dstp, onest)

    wss = jnp.stack([W_self1, W_self2])
    wns = jnp.stack([W_neigh1, W_neigh2])
    bss = jnp.stack([b1.reshape(1, D), b2.reshape(1, D)])
    flags = jnp.array([[[1.0]], [[0.0]]], dtype=jnp.float32)

    def body(hs, per):
        wsi, wni, bi, fl = per
        (summed,) = _sc_agg(hs, srcs, dstp)
        o1, o2 = _tc_layer(hs, hs, summed, summed, deg, deg, wsi, wni, bi, fl)
        return jnp.concatenate([o1, o2], axis=0), None

    hs_final, _ = lax.scan(body, xs, (wss, wns, bss, flags))
    return jnp.concatenate([hs_final[:N], hs_final[NP:NP + N]], axis=1)


# speedup vs baseline: 2.3214x; 1.1967x over previous
import functools

import jax
import jax.numpy as jnp
from jax import lax
from jax.experimental import pallas as pl
from jax.experimental.pallas import tpu as pltpu
from jax.experimental.pallas import tpu_sc as plsc

N = 10000
NP = 10240
E = 160000
D = 256
H = 128
NC = 2
NS = 16
EP = 163840
EPS = EP // NS
C = 80
NCHUNK = EPS // C
RPS = NP // NS
ZR = 64
DW = 16

_mesh = plsc.VectorSubcoreMesh(core_axis_name="c", subcore_axis_name="s",
                               num_cores=NC, num_subcores=NS)


def _sc_agg_body(xs, srcs3, dst4, summed,
                 srcidx_all, dstidx_all, rows0, rows1,
                 semg0, semg1, sems0, sems1, acc):
    c = lax.axis_index("c")
    s = lax.axis_index("s")

    rows = (rows0, rows1)
    semg = (semg0, semg1)
    sems = (sems0, sems1)

    zero16 = jnp.zeros((16,), jnp.float32)

    def fill(i, _):
        for j in range(H // 16):
            rows0[i, pl.ds(j * 16, 16)] = zero16
        return 0
    lax.fori_loop(0, C, fill, 0)

    pltpu.sync_copy(srcs3.at[c, s], srcidx_all)
    pltpu.sync_copy(dst4.at[s], dstidx_all)

    row0 = s * RPS
    for t in range(RPS // C):
        pltpu.sync_copy(rows0, acc.at[pl.ds(row0 + t * C, C)])
    plsc.subcore_barrier()

    def gather(k, b):
        pltpu.async_copy(xs.at[srcidx_all.at[pl.ds(k * C, C)]], rows[b],
                         semg[b])

    def gwait(b):
        pltpu.make_async_copy(xs.at[pl.ds(0, C)], rows[b], semg[b]).wait()

    def swait(b):
        pltpu.make_async_copy(xs.at[pl.ds(0, C)], rows[b], sems[b]).wait()

    gather(0, 0)

    def pair(k2, _):
        for b in range(2):
            k = k2 * 2 + b

            @pl.when(k + 1 < NCHUNK)
            def _():
                @pl.when(k >= 1)
                def _():
                    swait(1 - b)
                gather(k + 1, 1 - b)
            gwait(b)
            pltpu.async_copy(rows[b], acc.at[dstidx_all.at[k]], sems[b],
                             add=True)
        return 0
    lax.fori_loop(0, NCHUNK // 2, pair, 0)
    for b in range(2):
        swait(b)
    plsc.subcore_barrier()

    for t in range(RPS // C):
        r = row0 + t * C
        pltpu.sync_copy(acc.at[pl.ds(r, C)], rows0)
        pltpu.sync_copy(rows0, summed.at[pl.ds(c * NP + r, C)])


_sc_agg = pl.kernel(
    _sc_agg_body,
    out_type=(jax.ShapeDtypeStruct((NC * NP, H), jnp.float32),),
    mesh=_mesh,
    scratch_types=[
        pltpu.VMEM((EPS,), jnp.int32),
        pltpu.VMEM((NCHUNK, C), jnp.int32),
        pltpu.VMEM((C, H), jnp.float32),
        pltpu.VMEM((C, H), jnp.float32),
        pltpu.SemaphoreType.DMA,
        pltpu.SemaphoreType.DMA,
        pltpu.SemaphoreType.DMA,
        pltpu.SemaphoreType.DMA,
        pltpu.VMEM_SHARED((NP, H), jnp.float32),
    ],
)


QR = NP // 8
EPD = EP // (NC * NS)
NCHD = EPD // C
DRPS = QR // NS


def _sc_deg_body(dstp, onest, deg,
                 dstidx0, dstidx1, qidx0, qidx1, rows0, rows1,
                 semg, sems0, sems1, degacc, onessp):
    c = lax.axis_index("c")
    s = lax.axis_index("s")

    dstidx = (dstidx0, dstidx1)
    qidx = (qidx0, qidx1)
    rows = (rows0, rows1)
    sems = (sems0, sems1)

    zero16 = jnp.zeros((16,), jnp.float32)

    def fill(i, _):
        for j in range(H // 16):
            rows0[i, pl.ds(j * 16, 16)] = zero16
        return 0
    lax.fori_loop(0, C, fill, 0)

    row0 = s * DRPS
    pltpu.sync_copy(rows0.at[pl.ds(0, DRPS)], degacc.at[pl.ds(row0, DRPS)])
    @pl.when(s == 0)
    def _():
        pltpu.sync_copy(onest, onessp)
    plsc.subcore_barrier()

    ebase = (c * NS + s) * EPD

    def half(k, b):
        @pl.when(k >= 2)
        def _():
            pltpu.make_async_copy(deg.at[pl.ds(0, C)], rows[b], sems[b]).wait()
        pltpu.sync_copy(dstp.at[pl.ds(ebase + k * C, C)], dstidx[b])
        for j in range(C // 16):
            d = dstidx[b][pl.ds(j * 16, 16)]
            qidx[b][pl.ds(j * 16, 16)] = jax.lax.shift_right_logical(d, 3)
            dstidx[b][pl.ds(j * 16, 16)] = jnp.bitwise_and(d, 7)
        pltpu.async_copy(onessp.at[dstidx[b]], rows[b], semg).wait()
        pltpu.async_copy(rows[b], degacc.at[qidx[b]], sems[b], add=True)

    def chunk2(k2, _):
        half(k2 * 2, 0)
        half(k2 * 2 + 1, 1)
        return 0
    lax.fori_loop(0, NCHD // 2, chunk2, 0)
    for b in range(2):
        pltpu.make_async_copy(deg.at[pl.ds(0, C)], rows[b], sems[b]).wait()
    plsc.subcore_barrier()

    pltpu.sync_copy(degacc.at[pl.ds(row0, DRPS)], rows0.at[pl.ds(0, DRPS)])
    pltpu.sync_copy(rows0.at[pl.ds(0, DRPS)], deg.at[pl.ds(c * QR + row0, DRPS)])


_sc_deg = pl.kernel(
    _sc_deg_body,
    out_type=(jax.ShapeDtypeStruct((NC * QR, H), jnp.float32),),
    mesh=_mesh,
    scratch_types=[
        pltpu.VMEM((C,), jnp.int32),
        pltpu.VMEM((C,), jnp.int32),
        pltpu.VMEM((C,), jnp.int32),
        pltpu.VMEM((C,), jnp.int32),
        pltpu.VMEM((C, H), jnp.float32),
        pltpu.VMEM((C, H), jnp.float32),
        pltpu.SemaphoreType.DMA,
        pltpu.SemaphoreType.DMA,
        pltpu.SemaphoreType.DMA,
        pltpu.VMEM_SHARED((QR, H), jnp.float32),
        pltpu.VMEM_SHARED((8, H), jnp.float32),
    ],
)


R = 512
G = NP // R


def _tc_body(xa, xb, sa, sb, d0, d1, ws, wn, b, flag, o1, o2):
    dsum = d0[...] + d1[...]
    degn = dsum.reshape(R // 8, 8, 16)[:, :, 0].reshape(R, 1)
    invd = 1.0 / jnp.maximum(degn, 1.0)
    f32 = jnp.float32
    h = (jnp.dot(xa[...], ws[0:H, :], preferred_element_type=f32)
         + jnp.dot(xb[...], ws[H:D, :], preferred_element_type=f32)
         + jnp.dot(sa[...] * invd, wn[0:H, :], preferred_element_type=f32)
         + jnp.dot(sb[...] * invd, wn[H:D, :], preferred_element_type=f32)
         + b[...])
    h = jnp.where(flag[0, 0] > 0.5, jnp.maximum(h, 0.0), h)
    o1[...] = h[:, 0:H]
    o2[...] = h[:, H:D]


_tc_layer = pl.pallas_call(
    _tc_body,
    grid=(G,),
    in_specs=[
        pl.BlockSpec((R, H), lambda i: (i, 0)),
        pl.BlockSpec((R, H), lambda i: (i + G, 0)),
        pl.BlockSpec((R, H), lambda i: (i, 0)),
        pl.BlockSpec((R, H), lambda i: (i + G, 0)),
        pl.BlockSpec((R // 8, H), lambda i: (i, 0)),
        pl.BlockSpec((R // 8, H), lambda i: (i + G, 0)),
        pl.BlockSpec((D, D), lambda i: (0, 0)),
        pl.BlockSpec((D, D), lambda i: (0, 0)),
        pl.BlockSpec((1, D), lambda i: (0, 0)),
        pl.BlockSpec((1, 1), lambda i: (0, 0)),
    ],
    out_specs=(pl.BlockSpec((R, H), lambda i: (i, 0)),
               pl.BlockSpec((R, H), lambda i: (i, 0))),
    out_shape=(jax.ShapeDtypeStruct((NP, H), jnp.float32),
               jax.ShapeDtypeStruct((NP, H), jnp.float32)),
)


@jax.jit
def kernel(x, edge_index, W_self1, W_neigh1, b1, W_self2, W_neigh2, b2):
    xp = jnp.pad(x, ((0, NP - N), (0, 0)))
    xs = xp.reshape(NP, NC, H).transpose(1, 0, 2).reshape(NC * NP, H)
    src = edge_index[0]
    dst = edge_index[1]
    srcp = jnp.pad(src, (0, EP - E), constant_values=NP - 1)
    srcs3 = (jnp.concatenate([srcp, srcp + NP]).astype(jnp.int32)
             .reshape(NC, NS, EPS))
    dstp = jnp.pad(dst, (0, EP - E), constant_values=NP - 1).astype(jnp.int32)
    dst4 = dstp.reshape(NS, NCHUNK, C)
    onest = jnp.repeat(jnp.eye(8, dtype=jnp.float32), H // 8, axis=1)
    (deg,) = _sc_deg(dstp, onest)

    wss = jnp.stack([W_self1, W_self2])
    wns = jnp.stack([W_neigh1, W_neigh2])
    bss = jnp.stack([b1.reshape(1, D), b2.reshape(1, D)])
    flags = jnp.array([[[1.0]], [[0.0]]], dtype=jnp.float32)

    def body(hs, per):
        wsi, wni, bi, fl = per
        (summed,) = _sc_agg(hs, srcs3, dst4)
        o1, o2 = _tc_layer(hs, hs, summed, summed, deg, deg, wsi, wni, bi, fl)
        return jnp.concatenate([o1, o2], axis=0), None

    hs_final, _ = lax.scan(body, xs, (wss, wns, bss, flags))
    return jnp.concatenate([hs_final[:N], hs_final[NP:NP + N]], axis=1)
